# Initial kernel scaffold; baseline (speedup 1.0000x reference)
import jax
import jax.numpy as jnp
from jax.experimental import pallas as pl


def kernel(node_feats, g_feats, graph_ids, W1, b1, W2, b2):
    raise NotImplementedError("write your pallas kernel here")


# speedup vs baseline: 6.4404x; 6.4404x over previous
import functools
import math

import jax
import jax.numpy as jnp
from jax.experimental import pallas as pl
from jax.experimental.pallas import tpu as pltpu


def _pool_kernel(b1_ref, x_ref, ids_ref, gf_ref, w1a_ref, w1b_ref, w2_ref,
                 b2_ref, out_ref, u_ref, d_ref, s_ref, *, nb, G, Gp):
    i = pl.program_id(0)
    B = x_ref.shape[0]
    F = x_ref.shape[1]

    @pl.when(i == 0)
    def _init():
        u_ref[...] = jnp.zeros_like(u_ref)
        d_ref[...] = jnp.zeros_like(d_ref)
        s = jax.lax.dot_general(
            jnp.maximum(gf_ref[...], 0.0), w1a_ref[...],
            (((1,), (0,)), ((), ())), preferred_element_type=jnp.float32)
        s_ref[0:G, :] = s
        s_ref[G:Gp, :] = jnp.zeros((Gp - G, 1), jnp.float32)

    x = x_ref[...]
    ids = ids_ref[0, 0, :]
    giota = jax.lax.broadcasted_iota(jnp.int32, (Gp, B), 0)
    M = (giota == ids[None, :]).astype(jnp.float32)

    t = jax.lax.dot_general(x, w1b_ref[...], (((1,), (0,)), ((), ())),
                            preferred_element_type=jnp.float32)
    sg = jax.lax.dot_general(M, s_ref[...], (((0,), (0,)), ((), ())),
                             preferred_element_type=jnp.float32)
    z = sg + t + b1_ref[0, 0]
    z = jnp.where(z >= 0, z, 0.01 * z)
    ez = jnp.exp(z)

    y = x * ez
    u_ref[...] += jax.lax.dot_general(M, y, (((1,), (0,)), ((), ())),
                                      preferred_element_type=jnp.float32)
    d_ref[...] += jax.lax.dot_general(M, ez, (((1,), (0,)), ((), ())),
                                      preferred_element_type=jnp.float32)

    @pl.when(i == nb - 1)
    def _final():
        d = d_ref[...]
        dsafe = jnp.where(d > 0, d, 1.0)
        S = u_ref[...] / dsafe
        rep = jax.lax.dot_general(S, w2_ref[...], (((1,), (0,)), ((), ())),
                                  preferred_element_type=jnp.float32)
        rep = rep + jnp.where(d > 0, 1.0, 0.0) * b2_ref[...]
        out_ref[...] = rep[0:G, :]


@jax.jit
def kernel(node_feats, g_feats, graph_ids, W1, b1, W2, b2):
    V, F = node_feats.shape
    G = g_feats.shape[0]
    B = 512
    nb = math.ceil(V / B)
    Vp = nb * B
    Gp = ((G + 1 + 7) // 8) * 8

    x = jnp.pad(node_feats, ((0, Vp - V), (0, 0)))
    ids = jnp.pad(graph_ids, (0, Vp - V), constant_values=G)
    ids3 = ids.reshape(nb, 1, B)
    w1a = W1[:F]
    w1b = W1[F:]
    b1r = b1.reshape(1, 1)
    b2r = b2.reshape(1, F)

    grid = (nb,)
    out = pl.pallas_call(
        functools.partial(_pool_kernel, nb=nb, G=G, Gp=Gp),
        grid_spec=pltpu.PrefetchScalarGridSpec(
            num_scalar_prefetch=0,
            grid=grid,
            in_specs=[
                pl.BlockSpec(memory_space=pltpu.SMEM),
                pl.BlockSpec((B, F), lambda i: (i, 0)),
                pl.BlockSpec((1, 1, B), lambda i: (i, 0, 0)),
                pl.BlockSpec((G, F), lambda i: (0, 0)),
                pl.BlockSpec((F, 1), lambda i: (0, 0)),
                pl.BlockSpec((F, 1), lambda i: (0, 0)),
                pl.BlockSpec((F, F), lambda i: (0, 0)),
                pl.BlockSpec((1, F), lambda i: (0, 0)),
            ],
            out_specs=pl.BlockSpec((G, F), lambda i: (0, 0)),
            scratch_shapes=[
                pltpu.VMEM((Gp, F), jnp.float32),
                pltpu.VMEM((Gp, 1), jnp.float32),
                pltpu.VMEM((Gp, 1), jnp.float32),
            ],
        ),
        out_shape=jax.ShapeDtypeStruct((G, F), jnp.float32),
    )(b1r, x, ids3, g_feats, w1a, w1b, W2, b2r)
    return out
